# Initial kernel scaffold; baseline (speedup 1.0000x reference)
import jax
import jax.numpy as jnp
from jax.experimental import pallas as pl


def kernel(x, edge_index, edge_weight, W1_l, W1_r, b1, W3_l, W3_r, b3):
    raise NotImplementedError("write your pallas kernel here")


# speedup vs baseline: 6.8404x; 6.8404x over previous
import jax
import jax.numpy as jnp
from jax import lax
from jax.experimental import pallas as pl
from jax.experimental.pallas import tpu as pltpu
from jax.experimental.pallas import tpu_sc as plsc

NC = 2
NS = 16
K = 80
BLK = 4000


def _make_edge_pass(n_nodes, n_pad, emb, n_edges, with_deg):
    e_per_tile = n_edges // (NC * NS)
    assert e_per_tile * NC * NS == n_edges and e_per_tile % K == 0
    assert emb == 8 and 16 % emb == 0
    n_chunks = e_per_tile // K
    rows_per_tile = n_pad // NS
    assert rows_per_tile * NS == n_pad and rows_per_tile % 8 == 0

    mesh = plsc.VectorSubcoreMesh(
        core_axis_name="c", subcore_axis_name="s",
        num_cores=NC, num_subcores=NS)

    out_type = [jax.ShapeDtypeStruct((NC, n_pad, emb), jnp.float32)]
    scratch = [
        pltpu.VMEM((K,), jnp.int32),
        pltpu.VMEM((K,), jnp.int32),
        pltpu.VMEM((K,), jnp.float32),
        pltpu.VMEM((K, emb), jnp.float32),
        pltpu.VMEM_SHARED((n_pad, emb), jnp.float32),
        pltpu.SemaphoreType.DMA,
    ]
    if with_deg:
        out_type.append(jax.ShapeDtypeStruct((NC, n_nodes), jnp.float32))
        scratch.append(pltpu.VMEM((K,), jnp.float32))
        scratch.append(pltpu.VMEM_SHARED((n_nodes,), jnp.float32))

    def body(table, src, dst, w, zeros2, *rest):
        if with_deg:
            (zeros1, agg_out, deg_out, src_v, dst_v, w_v, rows_v,
             acc_sh, sem, ones_v, deg_sh) = rest
        else:
            (agg_out, src_v, dst_v, w_v, rows_v, acc_sh, sem) = rest

        c = lax.axis_index("c")
        s = lax.axis_index("s")
        base_e = (c * NS + s) * e_per_tile
        r0 = s * rows_per_tile

        pltpu.sync_copy(zeros2.at[pl.ds(r0, rows_per_tile)],
                        acc_sh.at[pl.ds(r0, rows_per_tile)])
        if with_deg:
            @pl.when(s == 0)
            def _():
                pltpu.sync_copy(zeros1, deg_sh)
            for i in range(K // 16):
                ones_v[pl.ds(i * 16, 16)] = jnp.full((16,), 1.0, jnp.float32)
        plsc.subcore_barrier()

        @pl.loop(0, n_chunks)
        def _chunk(j):
            e0 = base_e + j * K
            pltpu.sync_copy(src.at[pl.ds(e0, K)], src_v)
            pltpu.sync_copy(dst.at[pl.ds(e0, K)], dst_v)
            pltpu.sync_copy(w.at[pl.ds(e0, K)], w_v)
            pltpu.async_copy(table.at[src_v], rows_v, sem).wait()
            lane_row = lax.shift_right_logical(lax.iota(jnp.int32, 16), 3)
            lane_col = lax.iota(jnp.int32, 16) & 7
            for i in range(K // 2):
                ridx = lane_row + (2 * i)
                wv = plsc.load_gather(w_v, [ridx])
                v = plsc.load_gather(rows_v, [ridx, lane_col])
                plsc.store_scatter(rows_v, [ridx, lane_col], v * wv)
            pltpu.sync_copy(rows_v, acc_sh.at[dst_v], add=True)
            if with_deg:
                pltpu.sync_copy(ones_v, deg_sh.at[dst_v], add=True)

        plsc.subcore_barrier()
        pltpu.sync_copy(acc_sh.at[pl.ds(r0, rows_per_tile)],
                        agg_out.at[c, pl.ds(r0, rows_per_tile)])
        if with_deg:
            @pl.when(s == 0)
            def _():
                pltpu.sync_copy(deg_sh, deg_out.at[c])

    return pl.kernel(body, out_type=out_type, mesh=mesh,
                     scratch_types=scratch,
                     compiler_params=pltpu.CompilerParams(
                         use_tc_tiling_on_sc=False,
                         needs_layout_passes=False))


def _proj1_body(x_ref, wl_ref, wr_ref, b_ref, xl_ref, xr_ref):
    xb = x_ref[...]
    xl_ref[...] = jnp.dot(xb, wl_ref[...], preferred_element_type=jnp.float32)
    xr_ref[...] = (jnp.dot(xb, wr_ref[...], preferred_element_type=jnp.float32)
                   + b_ref[...])


def _mid_body(aggp_ref, degp_ref, xr_ref, wl_ref, wr_ref, b_ref,
              hl_ref, hr_ref):
    deg = jnp.clip(degp_ref[0] + degp_ref[1], 1.0, None)
    agg = (aggp_ref[0] + aggp_ref[1]) / deg
    h = jnp.maximum(agg + xr_ref[...], 0.0)
    hl_ref[...] = jnp.dot(h, wl_ref[...], preferred_element_type=jnp.float32)
    hr_ref[...] = (jnp.dot(h, wr_ref[...], preferred_element_type=jnp.float32)
                   + b_ref[...])


def _out_body(aggp_ref, degp_ref, hr_ref, o_ref):
    deg = jnp.clip(degp_ref[0] + degp_ref[1], 1.0, None)
    z = (aggp_ref[0] + aggp_ref[1]) / deg + hr_ref[...]
    m = jnp.max(z, axis=1, keepdims=True)
    e = jnp.exp(z - m)
    o_ref[...] = (z - m) - jnp.log(jnp.sum(e, axis=1, keepdims=True))


def kernel(x, edge_index, edge_weight, W1_l, W1_r, b1, W3_l, W3_r, b3):
    n, fin = x.shape
    emb = W1_l.shape[0]
    n_edges = edge_weight.shape[0]
    src = edge_index[0]
    dst = edge_index[1]
    rpt = -(-(n // NS) // 8) * 8
    n_pad = rpt * NS
    zeros2 = jnp.zeros((n_pad, emb), jnp.float32)
    zeros1 = jnp.zeros((n,), jnp.float32)
    grid = (n // BLK,)

    full = lambda shape: pl.BlockSpec(shape, lambda i: (0,) * len(shape))
    rows2 = pl.BlockSpec((BLK, fin), lambda i: (i, 0))
    rows_e = pl.BlockSpec((BLK, emb), lambda i: (i, 0))
    part3 = pl.BlockSpec((NC, BLK, emb), lambda i: (0, i, 0))
    part_deg = pl.BlockSpec((NC, BLK, 1), lambda i: (0, i, 0))
    f32 = lambda shape: jax.ShapeDtypeStruct(shape, jnp.float32)

    xl, xr = pl.pallas_call(
        _proj1_body, grid=grid,
        in_specs=[rows2, full((fin, emb)), full((fin, emb)), full((1, emb))],
        out_specs=[rows_e, rows_e],
        out_shape=[f32((n, emb)), f32((n, emb))],
    )(x, W1_l.T, W1_r.T, b1.reshape(1, emb))

    edge_pass_deg = _make_edge_pass(n, n_pad, emb, n_edges, with_deg=True)
    aggp1, degp = edge_pass_deg(xl, src, dst, edge_weight, zeros2, zeros1)
    aggp1 = aggp1[:, :n]
    degp3 = degp.reshape(NC, n, 1)

    hl, hr = pl.pallas_call(
        _mid_body, grid=grid,
        in_specs=[part3, part_deg, rows_e,
                  full((emb, emb)), full((emb, emb)), full((1, emb))],
        out_specs=[rows_e, rows_e],
        out_shape=[f32((n, emb)), f32((n, emb))],
    )(aggp1, degp3, xr, W3_l.T, W3_r.T, b3.reshape(1, emb))

    edge_pass = _make_edge_pass(n, n_pad, emb, n_edges, with_deg=False)
    aggp2, = edge_pass(hl, src, dst, edge_weight, zeros2)
    aggp2 = aggp2[:, :n]

    out = pl.pallas_call(
        _out_body, grid=grid,
        in_specs=[part3, part_deg, rows_e],
        out_specs=rows_e,
        out_shape=f32((n, emb)),
    )(aggp2, degp3, hr)
    return out


# speedup vs baseline: 13.4127x; 1.9608x over previous
import jax
import jax.numpy as jnp
from jax import lax
from jax.experimental import pallas as pl
from jax.experimental.pallas import tpu as pltpu
from jax.experimental.pallas import tpu_sc as plsc

NC = 2
NS = 16
K = 80
B = 10
BLK = 4000


def _make_edge_pass(n_nodes, n_pad, emb, n_edges, with_deg):
    e_per_tile = n_edges // (NC * NS)
    assert e_per_tile * NC * NS == n_edges and e_per_tile % (B * K) == 0
    assert emb == 8 and 16 % emb == 0
    n_blocks = e_per_tile // (B * K)
    rows_per_tile = n_pad // NS
    assert rows_per_tile * NS == n_pad and rows_per_tile % 8 == 0

    mesh = plsc.VectorSubcoreMesh(
        core_axis_name="c", subcore_axis_name="s",
        num_cores=NC, num_subcores=NS)

    out_type = [jax.ShapeDtypeStruct((NC, n_pad, emb), jnp.float32)]
    scratch = [
        pltpu.VMEM((B * K,), jnp.int32),
        pltpu.VMEM((B, K), jnp.int32),
        pltpu.VMEM((B * K,), jnp.float32),
        pltpu.VMEM((K, emb), jnp.float32),
        pltpu.VMEM((K, emb), jnp.float32),
        pltpu.VMEM_SHARED((n_pad, emb), jnp.float32),
        pltpu.SemaphoreType.DMA,
        pltpu.SemaphoreType.DMA,
        pltpu.SemaphoreType.DMA,
        pltpu.SemaphoreType.DMA,
        pltpu.SemaphoreType.DMA,
    ]
    if with_deg:
        out_type.append(jax.ShapeDtypeStruct((NC, n_nodes), jnp.float32))
        scratch.append(pltpu.VMEM((K,), jnp.float32))
        scratch.append(pltpu.VMEM_SHARED((n_nodes,), jnp.float32))
        scratch.append(pltpu.SemaphoreType.DMA)

    def body(table, src, dst2, w, zeros2, *rest):
        if with_deg:
            (zeros1, agg_out, deg_out, src_blk, dst_blk, w_blk,
             rows_a, rows_b, acc_sh, bsem, gsem_a, gsem_b, ssem_a, ssem_b,
             ones_v, deg_sh, dsem) = rest
        else:
            (agg_out, src_blk, dst_blk, w_blk,
             rows_a, rows_b, acc_sh, bsem, gsem_a, gsem_b,
             ssem_a, ssem_b) = rest

        c = lax.axis_index("c")
        s = lax.axis_index("s")
        base_e = (c * NS + s) * e_per_tile
        r0 = s * rows_per_tile

        pltpu.sync_copy(zeros2.at[pl.ds(r0, rows_per_tile)],
                        acc_sh.at[pl.ds(r0, rows_per_tile)])
        if with_deg:
            @pl.when(s == 0)
            def _():
                pltpu.sync_copy(zeros1, deg_sh)
            for i in range(K // 16):
                ones_v[pl.ds(i * 16, 16)] = jnp.full((16,), 1.0, jnp.float32)
        plsc.subcore_barrier()

        rows = (rows_a, rows_b)
        gsem = (gsem_a, gsem_b)
        ssem = (ssem_a, ssem_b)
        lane_row = lax.shift_right_logical(lax.iota(jnp.int32, 16), 3)
        lane_col = lax.iota(jnp.int32, 16) & 7

        @pl.loop(0, n_blocks)
        def _block(blk):
            e0 = base_e + blk * (B * K)
            row0 = lax.div(e0, K)
            ld_s = pltpu.async_copy(src.at[pl.ds(e0, B * K)], src_blk, bsem)
            ld_d = pltpu.async_copy(dst2.at[pl.ds(row0, B)], dst_blk, bsem)
            ld_w = pltpu.async_copy(w.at[pl.ds(e0, B * K)], w_blk, bsem)
            ld_s.wait(); ld_d.wait(); ld_w.wait()

            g_desc = [None, None]
            s_desc = [None, None]
            d_desc = [None]
            g_desc[0] = pltpu.async_copy(
                table.at[src_blk.at[pl.ds(0, K)]], rows[0], gsem[0])
            for b in range(B):
                p = b & 1
                g_desc[p].wait()
                for i in range(K // 2):
                    ridx = lane_row + (2 * i)
                    wv = plsc.load_gather(w_blk, [ridx + (b * K)])
                    v = plsc.load_gather(rows[p], [ridx, lane_col])
                    plsc.store_scatter(rows[p], [ridx, lane_col], v * wv)
                s_desc[p] = pltpu.async_copy(
                    rows[p], acc_sh.at[dst_blk.at[b]], ssem[p], add=True)
                if with_deg:
                    if d_desc[0] is not None:
                        d_desc[0].wait()
                    d_desc[0] = pltpu.async_copy(
                        ones_v, deg_sh.at[dst_blk.at[b]], dsem, add=True)
                if b + 1 < B:
                    q = 1 - p
                    if s_desc[q] is not None:
                        s_desc[q].wait()
                    g_desc[q] = pltpu.async_copy(
                        table.at[src_blk.at[pl.ds((b + 1) * K, K)]],
                        rows[q], gsem[q])
            s_desc[0].wait()
            s_desc[1].wait()
            if with_deg:
                d_desc[0].wait()

        plsc.subcore_barrier()
        pltpu.sync_copy(acc_sh.at[pl.ds(r0, rows_per_tile)],
                        agg_out.at[c, pl.ds(r0, rows_per_tile)])
        if with_deg:
            @pl.when(s == 0)
            def _():
                pltpu.sync_copy(deg_sh, deg_out.at[c])

    return pl.kernel(body, out_type=out_type, mesh=mesh,
                     scratch_types=scratch,
                     compiler_params=pltpu.CompilerParams(
                         use_tc_tiling_on_sc=False,
                         needs_layout_passes=False))


def _proj1_body(x_ref, wl_ref, wr_ref, b_ref, xl_ref, xr_ref):
    xb = x_ref[...]
    xl_ref[...] = jnp.dot(xb, wl_ref[...], preferred_element_type=jnp.float32)
    xr_ref[...] = (jnp.dot(xb, wr_ref[...], preferred_element_type=jnp.float32)
                   + b_ref[...])


def _mid_body(aggp_ref, degp_ref, xr_ref, wl_ref, wr_ref, b_ref,
              hl_ref, hr_ref):
    deg = jnp.clip(degp_ref[0] + degp_ref[1], 1.0, None)
    agg = (aggp_ref[0] + aggp_ref[1]) / deg
    h = jnp.maximum(agg + xr_ref[...], 0.0)
    hl_ref[...] = jnp.dot(h, wl_ref[...], preferred_element_type=jnp.float32)
    hr_ref[...] = (jnp.dot(h, wr_ref[...], preferred_element_type=jnp.float32)
                   + b_ref[...])


def _out_body(aggp_ref, degp_ref, hr_ref, o_ref):
    deg = jnp.clip(degp_ref[0] + degp_ref[1], 1.0, None)
    z = (aggp_ref[0] + aggp_ref[1]) / deg + hr_ref[...]
    m = jnp.max(z, axis=1, keepdims=True)
    e = jnp.exp(z - m)
    o_ref[...] = (z - m) - jnp.log(jnp.sum(e, axis=1, keepdims=True))


def kernel(x, edge_index, edge_weight, W1_l, W1_r, b1, W3_l, W3_r, b3):
    n, fin = x.shape
    emb = W1_l.shape[0]
    n_edges = edge_weight.shape[0]
    src = edge_index[0]
    dst2 = edge_index[1].reshape(n_edges // K, K)
    rpt = -(-(n // NS) // 8) * 8
    n_pad = rpt * NS
    zeros2 = jnp.zeros((n_pad, emb), jnp.float32)
    zeros1 = jnp.zeros((n,), jnp.float32)
    grid = (n // BLK,)

    full = lambda shape: pl.BlockSpec(shape, lambda i: (0,) * len(shape))
    rows2 = pl.BlockSpec((BLK, fin), lambda i: (i, 0))
    rows_e = pl.BlockSpec((BLK, emb), lambda i: (i, 0))
    part3 = pl.BlockSpec((NC, BLK, emb), lambda i: (0, i, 0))
    part_deg = pl.BlockSpec((NC, BLK, 1), lambda i: (0, i, 0))
    f32 = lambda shape: jax.ShapeDtypeStruct(shape, jnp.float32)

    xl, xr = pl.pallas_call(
        _proj1_body, grid=grid,
        in_specs=[rows2, full((fin, emb)), full((fin, emb)), full((1, emb))],
        out_specs=[rows_e, rows_e],
        out_shape=[f32((n, emb)), f32((n, emb))],
    )(x, W1_l.T, W1_r.T, b1.reshape(1, emb))

    edge_pass_deg = _make_edge_pass(n, n_pad, emb, n_edges, with_deg=True)
    aggp1, degp = edge_pass_deg(xl, src, dst2, edge_weight, zeros2, zeros1)
    aggp1 = aggp1[:, :n]
    degp3 = degp.reshape(NC, n, 1)

    hl, hr = pl.pallas_call(
        _mid_body, grid=grid,
        in_specs=[part3, part_deg, rows_e,
                  full((emb, emb)), full((emb, emb)), full((1, emb))],
        out_specs=[rows_e, rows_e],
        out_shape=[f32((n, emb)), f32((n, emb))],
    )(aggp1, degp3, xr, W3_l.T, W3_r.T, b3.reshape(1, emb))

    edge_pass = _make_edge_pass(n, n_pad, emb, n_edges, with_deg=False)
    aggp2, = edge_pass(hl, src, dst2, edge_weight, zeros2)
    aggp2 = aggp2[:, :n]

    out = pl.pallas_call(
        _out_body, grid=grid,
        in_specs=[part3, part_deg, rows_e],
        out_specs=rows_e,
        out_shape=f32((n, emb)),
    )(aggp2, degp3, hr)
    return out


# speedup vs baseline: 14.2967x; 1.0659x over previous
import jax
import jax.numpy as jnp
from jax import lax
from jax.experimental import pallas as pl
from jax.experimental.pallas import tpu as pltpu
from jax.experimental.pallas import tpu_sc as plsc

NC = 2
NS = 16
K = 80
B = 25
BLK = 4000


def _make_edge_pass(n_nodes, n_pad, emb, n_edges, with_deg):
    e_per_tile = n_edges // (NC * NS)
    assert e_per_tile * NC * NS == n_edges and e_per_tile % (B * K) == 0
    assert emb == 8 and 16 % emb == 0
    n_blocks = e_per_tile // (B * K)
    rows_per_tile = n_pad // NS
    assert rows_per_tile * NS == n_pad and rows_per_tile % 8 == 0

    mesh = plsc.VectorSubcoreMesh(
        core_axis_name="c", subcore_axis_name="s",
        num_cores=NC, num_subcores=NS)

    out_type = [jax.ShapeDtypeStruct((NC, n_pad, emb), jnp.float32)]
    scratch = [
        pltpu.VMEM((B * K,), jnp.int32),
        pltpu.VMEM((B, K), jnp.int32),
        pltpu.VMEM((B * K,), jnp.float32),
        pltpu.VMEM((K, emb), jnp.float32),
        pltpu.VMEM((K, emb), jnp.float32),
        pltpu.VMEM_SHARED((n_pad, emb), jnp.float32),
        pltpu.SemaphoreType.DMA,
        pltpu.SemaphoreType.DMA,
        pltpu.SemaphoreType.DMA,
        pltpu.SemaphoreType.DMA,
        pltpu.SemaphoreType.DMA,
    ]
    if with_deg:
        out_type.append(jax.ShapeDtypeStruct((NC, n_nodes), jnp.float32))
        scratch.append(pltpu.VMEM((K,), jnp.float32))
        scratch.append(pltpu.VMEM_SHARED((n_nodes,), jnp.float32))
        scratch.append(pltpu.SemaphoreType.DMA)

    def body(table, src, dst2, w, zeros2, *rest):
        if with_deg:
            (zeros1, agg_out, deg_out, src_blk, dst_blk, w_blk,
             rows_a, rows_b, acc_sh, bsem, gsem_a, gsem_b, ssem_a, ssem_b,
             ones_v, deg_sh, dsem) = rest
        else:
            (agg_out, src_blk, dst_blk, w_blk,
             rows_a, rows_b, acc_sh, bsem, gsem_a, gsem_b,
             ssem_a, ssem_b) = rest

        c = lax.axis_index("c")
        s = lax.axis_index("s")
        base_e = (c * NS + s) * e_per_tile
        r0 = s * rows_per_tile

        pltpu.sync_copy(zeros2.at[pl.ds(r0, rows_per_tile)],
                        acc_sh.at[pl.ds(r0, rows_per_tile)])
        if with_deg:
            @pl.when(s == 0)
            def _():
                pltpu.sync_copy(zeros1, deg_sh)
            for i in range(K // 16):
                ones_v[pl.ds(i * 16, 16)] = jnp.full((16,), 1.0, jnp.float32)
        plsc.subcore_barrier()

        rows = (rows_a, rows_b)
        gsem = (gsem_a, gsem_b)
        ssem = (ssem_a, ssem_b)
        lane_row = lax.shift_right_logical(lax.iota(jnp.int32, 16), 3)
        lane_col = lax.iota(jnp.int32, 16) & 7

        @pl.loop(0, n_blocks)
        def _block(blk):
            e0 = base_e + blk * (B * K)
            row0 = lax.div(e0, K)
            ld_s = pltpu.async_copy(src.at[pl.ds(e0, B * K)], src_blk, bsem)
            ld_d = pltpu.async_copy(dst2.at[pl.ds(row0, B)], dst_blk, bsem)
            ld_w = pltpu.async_copy(w.at[pl.ds(e0, B * K)], w_blk, bsem)
            ld_s.wait(); ld_d.wait(); ld_w.wait()

            g_desc = [None, None]
            s_desc = [None, None]
            d_desc = [None]
            g_desc[0] = pltpu.async_copy(
                table.at[src_blk.at[pl.ds(0, K)]], rows[0], gsem[0])
            for b in range(B):
                p = b & 1
                g_desc[p].wait()
                for i in range(K // 2):
                    ridx = lane_row + (2 * i)
                    wv = plsc.load_gather(w_blk, [ridx + (b * K)])
                    v = plsc.load_gather(rows[p], [ridx, lane_col])
                    plsc.store_scatter(rows[p], [ridx, lane_col], v * wv)
                s_desc[p] = pltpu.async_copy(
                    rows[p], acc_sh.at[dst_blk.at[b]], ssem[p], add=True)
                if with_deg:
                    if d_desc[0] is not None:
                        d_desc[0].wait()
                    d_desc[0] = pltpu.async_copy(
                        ones_v, deg_sh.at[dst_blk.at[b]], dsem, add=True)
                if b + 1 < B:
                    q = 1 - p
                    if s_desc[q] is not None:
                        s_desc[q].wait()
                    g_desc[q] = pltpu.async_copy(
                        table.at[src_blk.at[pl.ds((b + 1) * K, K)]],
                        rows[q], gsem[q])
            s_desc[0].wait()
            s_desc[1].wait()
            if with_deg:
                d_desc[0].wait()

        plsc.subcore_barrier()
        pltpu.sync_copy(acc_sh.at[pl.ds(r0, rows_per_tile)],
                        agg_out.at[c, pl.ds(r0, rows_per_tile)])
        if with_deg:
            @pl.when(s == 0)
            def _():
                pltpu.sync_copy(deg_sh, deg_out.at[c])

    return pl.kernel(body, out_type=out_type, mesh=mesh,
                     scratch_types=scratch,
                     compiler_params=pltpu.CompilerParams(
                         use_tc_tiling_on_sc=False,
                         needs_layout_passes=False))


def _proj1_body(x_ref, wl_ref, wr_ref, b_ref, xl_ref, xr_ref):
    xb = x_ref[...]
    xl_ref[...] = jnp.dot(xb, wl_ref[...], preferred_element_type=jnp.float32)
    xr_ref[...] = (jnp.dot(xb, wr_ref[...], preferred_element_type=jnp.float32)
                   + b_ref[...])


def _mid_body(aggp_ref, degp_ref, xr_ref, wl_ref, wr_ref, b_ref,
              hl_ref, hr_ref):
    deg = jnp.clip(degp_ref[0] + degp_ref[1], 1.0, None)
    agg = (aggp_ref[0] + aggp_ref[1]) / deg
    h = jnp.maximum(agg + xr_ref[...], 0.0)
    hl_ref[...] = jnp.dot(h, wl_ref[...], preferred_element_type=jnp.float32)
    hr_ref[...] = (jnp.dot(h, wr_ref[...], preferred_element_type=jnp.float32)
                   + b_ref[...])


def _out_body(aggp_ref, degp_ref, hr_ref, o_ref):
    deg = jnp.clip(degp_ref[0] + degp_ref[1], 1.0, None)
    z = (aggp_ref[0] + aggp_ref[1]) / deg + hr_ref[...]
    m = jnp.max(z, axis=1, keepdims=True)
    e = jnp.exp(z - m)
    o_ref[...] = (z - m) - jnp.log(jnp.sum(e, axis=1, keepdims=True))


def kernel(x, edge_index, edge_weight, W1_l, W1_r, b1, W3_l, W3_r, b3):
    n, fin = x.shape
    emb = W1_l.shape[0]
    n_edges = edge_weight.shape[0]
    src = edge_index[0]
    dst2 = edge_index[1].reshape(n_edges // K, K)
    rpt = -(-(n // NS) // 8) * 8
    n_pad = rpt * NS
    zeros2 = jnp.zeros((n_pad, emb), jnp.float32)
    zeros1 = jnp.zeros((n,), jnp.float32)
    grid = (n // BLK,)

    full = lambda shape: pl.BlockSpec(shape, lambda i: (0,) * len(shape))
    rows2 = pl.BlockSpec((BLK, fin), lambda i: (i, 0))
    rows_e = pl.BlockSpec((BLK, emb), lambda i: (i, 0))
    part3 = pl.BlockSpec((NC, BLK, emb), lambda i: (0, i, 0))
    part_deg = pl.BlockSpec((NC, BLK, 1), lambda i: (0, i, 0))
    f32 = lambda shape: jax.ShapeDtypeStruct(shape, jnp.float32)

    xl, xr = pl.pallas_call(
        _proj1_body, grid=grid,
        in_specs=[rows2, full((fin, emb)), full((fin, emb)), full((1, emb))],
        out_specs=[rows_e, rows_e],
        out_shape=[f32((n, emb)), f32((n, emb))],
    )(x, W1_l.T, W1_r.T, b1.reshape(1, emb))

    edge_pass_deg = _make_edge_pass(n, n_pad, emb, n_edges, with_deg=True)
    aggp1, degp = edge_pass_deg(xl, src, dst2, edge_weight, zeros2, zeros1)
    degp3 = degp.reshape(NC, n, 1)

    hl, hr = pl.pallas_call(
        _mid_body, grid=grid,
        in_specs=[part3, part_deg, rows_e,
                  full((emb, emb)), full((emb, emb)), full((1, emb))],
        out_specs=[rows_e, rows_e],
        out_shape=[f32((n, emb)), f32((n, emb))],
    )(aggp1, degp3, xr, W3_l.T, W3_r.T, b3.reshape(1, emb))

    edge_pass = _make_edge_pass(n, n_pad, emb, n_edges, with_deg=False)
    aggp2, = edge_pass(hl, src, dst2, edge_weight, zeros2)

    out = pl.pallas_call(
        _out_body, grid=grid,
        in_specs=[part3, part_deg, rows_e],
        out_specs=rows_e,
        out_shape=f32((n, emb)),
    )(aggp2, degp3, hr)
    return out


# speedup vs baseline: 20.0644x; 1.4034x over previous
import jax
import jax.numpy as jnp
from jax import lax
from jax.experimental import pallas as pl
from jax.experimental.pallas import tpu as pltpu
from jax.experimental.pallas import tpu_sc as plsc

NC = 2
NS = 16
K = 80
B = 25
BLK = 4000


def _make_edge_pass(n_nodes, n_pad, emb, n_edges, with_deg):
    e_per_tile = n_edges // (NC * NS)
    assert e_per_tile * NC * NS == n_edges and e_per_tile % (B * K) == 0
    assert emb == 8 and 16 % emb == 0
    n_blocks = e_per_tile // (B * K)
    rows_per_tile = n_pad // NS
    assert rows_per_tile * NS == n_pad and rows_per_tile % 8 == 0

    mesh = plsc.VectorSubcoreMesh(
        core_axis_name="c", subcore_axis_name="s",
        num_cores=NC, num_subcores=NS)

    out_type = [jax.ShapeDtypeStruct((NC, n_pad, emb), jnp.float32)]
    scratch = [
        pltpu.VMEM((B * K,), jnp.int32),
        pltpu.VMEM((B, K), jnp.int32),
        pltpu.VMEM((B * K,), jnp.float32),
        pltpu.VMEM((K, emb), jnp.float32),
        pltpu.VMEM((K, emb), jnp.float32),
        pltpu.VMEM((K, emb), jnp.float32),
        pltpu.VMEM((K, emb), jnp.float32),
        pltpu.VMEM_SHARED((n_pad, emb), jnp.float32),
        pltpu.SemaphoreType.DMA,
        pltpu.SemaphoreType.DMA,
        pltpu.SemaphoreType.DMA,
        pltpu.SemaphoreType.DMA,
        pltpu.SemaphoreType.DMA,
    ]
    if with_deg:
        out_type.append(jax.ShapeDtypeStruct((NC, n_nodes), jnp.float32))
        scratch.append(pltpu.VMEM((K,), jnp.float32))
        scratch.append(pltpu.VMEM_SHARED((n_nodes,), jnp.float32))
        scratch.append(pltpu.SemaphoreType.DMA)

    def body(table, src, dst2, w, zeros2, *rest):
        if with_deg:
            (zeros1, agg_out, deg_out, src_blk, dst_blk, w_blk,
             rows_a, rows_b, sc_a, sc_b, acc_sh, bsem, gsem_a, gsem_b,
             ssem_a, ssem_b, ones_v, deg_sh, dsem) = rest
        else:
            (agg_out, src_blk, dst_blk, w_blk,
             rows_a, rows_b, sc_a, sc_b, acc_sh, bsem, gsem_a, gsem_b,
             ssem_a, ssem_b) = rest

        c = lax.axis_index("c")
        s = lax.axis_index("s")
        base_e = (c * NS + s) * e_per_tile
        r0 = s * rows_per_tile

        pltpu.sync_copy(zeros2.at[pl.ds(r0, rows_per_tile)],
                        acc_sh.at[pl.ds(r0, rows_per_tile)])
        if with_deg:
            @pl.when(s == 0)
            def _():
                pltpu.sync_copy(zeros1, deg_sh)
            for i in range(K // 16):
                ones_v[pl.ds(i * 16, 16)] = jnp.full((16,), 1.0, jnp.float32)
        plsc.subcore_barrier()

        rows = (rows_a, rows_b)
        scrows = (sc_a, sc_b)
        gsem = (gsem_a, gsem_b)
        ssem = (ssem_a, ssem_b)
        lane_row = lax.shift_right_logical(lax.iota(jnp.int32, 16), 3)
        lane_col = lax.iota(jnp.int32, 16) & 7

        @pl.loop(0, n_blocks)
        def _block(blk):
            e0 = base_e + blk * (B * K)
            row0 = lax.div(e0, K)
            ld_s = pltpu.async_copy(src.at[pl.ds(e0, B * K)], src_blk, bsem)
            ld_d = pltpu.async_copy(dst2.at[pl.ds(row0, B)], dst_blk, bsem)
            ld_w = pltpu.async_copy(w.at[pl.ds(e0, B * K)], w_blk, bsem)
            ld_s.wait(); ld_d.wait(); ld_w.wait()

            g_desc = [None, None]
            s_desc = [None, None]
            d_desc = [None]
            g_desc[0] = pltpu.async_copy(
                table.at[src_blk.at[pl.ds(0, K)]], rows[0], gsem[0])
            for b in range(B):
                p = b & 1
                q = 1 - p
                if b + 1 < B:
                    g_desc[q] = pltpu.async_copy(
                        table.at[src_blk.at[pl.ds((b + 1) * K, K)]],
                        rows[q], gsem[q])
                g_desc[p].wait()
                if s_desc[p] is not None:
                    s_desc[p].wait()
                for i in range(K // 2):
                    ridx = lane_row + (2 * i)
                    wv = plsc.load_gather(w_blk, [ridx + (b * K)])
                    v = plsc.load_gather(rows[p], [ridx, lane_col])
                    plsc.store_scatter(scrows[p], [ridx, lane_col], v * wv)
                s_desc[p] = pltpu.async_copy(
                    scrows[p], acc_sh.at[dst_blk.at[b]], ssem[p], add=True)
                if with_deg:
                    if d_desc[0] is not None:
                        d_desc[0].wait()
                    d_desc[0] = pltpu.async_copy(
                        ones_v, deg_sh.at[dst_blk.at[b]], dsem, add=True)
            s_desc[0].wait()
            s_desc[1].wait()
            if with_deg:
                d_desc[0].wait()

        plsc.subcore_barrier()
        pltpu.sync_copy(acc_sh.at[pl.ds(r0, rows_per_tile)],
                        agg_out.at[c, pl.ds(r0, rows_per_tile)])
        if with_deg:
            @pl.when(s == 0)
            def _():
                pltpu.sync_copy(deg_sh, deg_out.at[c])

    return pl.kernel(body, out_type=out_type, mesh=mesh,
                     scratch_types=scratch,
                     compiler_params=pltpu.CompilerParams(
                         use_tc_tiling_on_sc=False,
                         needs_layout_passes=False))


def _proj1_body(x_ref, wl_ref, wr_ref, b_ref, xl_ref, xr_ref):
    xb = x_ref[...]
    xl_ref[...] = jnp.dot(xb, wl_ref[...], preferred_element_type=jnp.float32)
    xr_ref[...] = (jnp.dot(xb, wr_ref[...], preferred_element_type=jnp.float32)
                   + b_ref[...])


def _mid_body(aggp_ref, degp_ref, xr_ref, wl_ref, wr_ref, b_ref,
              hl_ref, hr_ref):
    deg = jnp.clip(degp_ref[0] + degp_ref[1], 1.0, None)
    agg = (aggp_ref[0] + aggp_ref[1]) / deg
    h = jnp.maximum(agg + xr_ref[...], 0.0)
    hl_ref[...] = jnp.dot(h, wl_ref[...], preferred_element_type=jnp.float32)
    hr_ref[...] = (jnp.dot(h, wr_ref[...], preferred_element_type=jnp.float32)
                   + b_ref[...])


def _out_body(aggp_ref, degp_ref, hr_ref, o_ref):
    deg = jnp.clip(degp_ref[0] + degp_ref[1], 1.0, None)
    z = (aggp_ref[0] + aggp_ref[1]) / deg + hr_ref[...]
    m = jnp.max(z, axis=1, keepdims=True)
    e = jnp.exp(z - m)
    o_ref[...] = (z - m) - jnp.log(jnp.sum(e, axis=1, keepdims=True))


def kernel(x, edge_index, edge_weight, W1_l, W1_r, b1, W3_l, W3_r, b3):
    n, fin = x.shape
    emb = W1_l.shape[0]
    n_edges = edge_weight.shape[0]
    src = edge_index[0]
    dst2 = edge_index[1].reshape(n_edges // K, K)
    rpt = -(-(n // NS) // 8) * 8
    n_pad = rpt * NS
    zeros2 = jnp.zeros((n_pad, emb), jnp.float32)
    zeros1 = jnp.zeros((n,), jnp.float32)
    grid = (n // BLK,)

    full = lambda shape: pl.BlockSpec(shape, lambda i: (0,) * len(shape))
    rows2 = pl.BlockSpec((BLK, fin), lambda i: (i, 0))
    rows_e = pl.BlockSpec((BLK, emb), lambda i: (i, 0))
    part3 = pl.BlockSpec((NC, BLK, emb), lambda i: (0, i, 0))
    part_deg = pl.BlockSpec((NC, BLK, 1), lambda i: (0, i, 0))
    f32 = lambda shape: jax.ShapeDtypeStruct(shape, jnp.float32)

    xl, xr = pl.pallas_call(
        _proj1_body, grid=grid,
        in_specs=[rows2, full((fin, emb)), full((fin, emb)), full((1, emb))],
        out_specs=[rows_e, rows_e],
        out_shape=[f32((n, emb)), f32((n, emb))],
    )(x, W1_l.T, W1_r.T, b1.reshape(1, emb))

    edge_pass_deg = _make_edge_pass(n, n_pad, emb, n_edges, with_deg=True)
    aggp1, degp = edge_pass_deg(xl, src, dst2, edge_weight, zeros2, zeros1)
    degp3 = degp.reshape(NC, n, 1)

    hl, hr = pl.pallas_call(
        _mid_body, grid=grid,
        in_specs=[part3, part_deg, rows_e,
                  full((emb, emb)), full((emb, emb)), full((1, emb))],
        out_specs=[rows_e, rows_e],
        out_shape=[f32((n, emb)), f32((n, emb))],
    )(aggp1, degp3, xr, W3_l.T, W3_r.T, b3.reshape(1, emb))

    edge_pass = _make_edge_pass(n, n_pad, emb, n_edges, with_deg=False)
    aggp2, = edge_pass(hl, src, dst2, edge_weight, zeros2)

    out = pl.pallas_call(
        _out_body, grid=grid,
        in_specs=[part3, part_deg, rows_e],
        out_specs=rows_e,
        out_shape=f32((n, emb)),
    )(aggp2, degp3, hr)
    return out


# speedup vs baseline: 21.5265x; 1.0729x over previous
import jax
import jax.numpy as jnp
from jax import lax
from jax.experimental import pallas as pl
from jax.experimental.pallas import tpu as pltpu
from jax.experimental.pallas import tpu_sc as plsc

NC = 2
NS = 16
K = 128
B = 16
BLK = 4000


def _make_edge_pass(n_nodes, n_pad, emb, n_edges, with_deg):
    e_per_tile = n_edges // (NC * NS)
    assert e_per_tile * NC * NS == n_edges and e_per_tile % (B * K) == 0
    assert emb == 8 and 16 % emb == 0
    n_blocks = e_per_tile // (B * K)
    rows_per_tile = n_pad // NS
    assert rows_per_tile * NS == n_pad and rows_per_tile % 8 == 0

    mesh = plsc.VectorSubcoreMesh(
        core_axis_name="c", subcore_axis_name="s",
        num_cores=NC, num_subcores=NS)

    out_type = [jax.ShapeDtypeStruct((NC, n_pad, emb), jnp.float32)]
    scratch = [
        pltpu.VMEM((B * K,), jnp.int32),
        pltpu.VMEM((B, K), jnp.int32),
        pltpu.VMEM((B * K,), jnp.float32),
        pltpu.VMEM((K, emb), jnp.float32),
        pltpu.VMEM((K, emb), jnp.float32),
        pltpu.VMEM((K, emb), jnp.float32),
        pltpu.VMEM((K, emb), jnp.float32),
        pltpu.VMEM_SHARED((n_pad, emb), jnp.float32),
        pltpu.SemaphoreType.DMA,
        pltpu.SemaphoreType.DMA,
        pltpu.SemaphoreType.DMA,
        pltpu.SemaphoreType.DMA,
        pltpu.SemaphoreType.DMA,
    ]
    if with_deg:
        out_type.append(jax.ShapeDtypeStruct((NC, n_pad), jnp.float32))
        scratch.append(pltpu.VMEM((K,), jnp.float32))
        scratch.append(pltpu.VMEM_SHARED((n_pad,), jnp.float32))
        scratch.append(pltpu.SemaphoreType.DMA)

    def body(table, src, dst2, w, zeros2, *rest):
        if with_deg:
            (zeros1, agg_out, deg_out, src_blk, dst_blk, w_blk,
             rows_a, rows_b, sc_a, sc_b, acc_sh, bsem, gsem_a, gsem_b,
             ssem_a, ssem_b, ones_v, deg_sh, dsem) = rest
        else:
            (agg_out, src_blk, dst_blk, w_blk,
             rows_a, rows_b, sc_a, sc_b, acc_sh, bsem, gsem_a, gsem_b,
             ssem_a, ssem_b) = rest

        c = lax.axis_index("c")
        s = lax.axis_index("s")
        base_e = (c * NS + s) * e_per_tile
        r0 = s * rows_per_tile

        pltpu.sync_copy(zeros2.at[pl.ds(r0, rows_per_tile)],
                        acc_sh.at[pl.ds(r0, rows_per_tile)])
        if with_deg:
            @pl.when(s == 0)
            def _():
                pltpu.sync_copy(zeros1, deg_sh)
            for i in range(K // 16):
                ones_v[pl.ds(i * 16, 16)] = jnp.full((16,), 1.0, jnp.float32)
        plsc.subcore_barrier()

        rows = (rows_a, rows_b)
        scrows = (sc_a, sc_b)
        gsem = (gsem_a, gsem_b)
        ssem = (ssem_a, ssem_b)
        lane_row = lax.shift_right_logical(lax.iota(jnp.int32, 16), 3)
        lane_col = lax.iota(jnp.int32, 16) & 7

        @pl.loop(0, n_blocks)
        def _block(blk):
            e0 = base_e + blk * (B * K)
            row0 = lax.div(e0, K)
            ld_s = pltpu.async_copy(src.at[pl.ds(e0, B * K)], src_blk, bsem)
            ld_d = pltpu.async_copy(dst2.at[pl.ds(row0, B)], dst_blk, bsem)
            ld_w = pltpu.async_copy(w.at[pl.ds(e0, B * K)], w_blk, bsem)
            ld_s.wait(); ld_d.wait(); ld_w.wait()

            g_desc = [None, None]
            s_desc = [None, None]
            d_desc = [None]
            g_desc[0] = pltpu.async_copy(
                table.at[src_blk.at[pl.ds(0, K)]], rows[0], gsem[0])
            for b in range(B):
                p = b & 1
                q = 1 - p
                if b + 1 < B:
                    g_desc[q] = pltpu.async_copy(
                        table.at[src_blk.at[pl.ds((b + 1) * K, K)]],
                        rows[q], gsem[q])
                g_desc[p].wait()
                if s_desc[p] is not None:
                    s_desc[p].wait()
                for i in range(K // 2):
                    ridx = lane_row + (2 * i)
                    wv = plsc.load_gather(w_blk, [ridx + (b * K)])
                    v = plsc.load_gather(rows[p], [ridx, lane_col])
                    plsc.store_scatter(scrows[p], [ridx, lane_col], v * wv)
                s_desc[p] = pltpu.async_copy(
                    scrows[p], acc_sh.at[dst_blk.at[b]], ssem[p], add=True)
                if with_deg:
                    if d_desc[0] is not None:
                        d_desc[0].wait()
                    d_desc[0] = pltpu.async_copy(
                        ones_v, deg_sh.at[dst_blk.at[b]], dsem, add=True)
            s_desc[0].wait()
            s_desc[1].wait()
            if with_deg:
                d_desc[0].wait()

        plsc.subcore_barrier()
        pltpu.sync_copy(acc_sh.at[pl.ds(r0, rows_per_tile)],
                        agg_out.at[c, pl.ds(r0, rows_per_tile)])
        if with_deg:
            @pl.when(s == 0)
            def _():
                pltpu.sync_copy(deg_sh, deg_out.at[c])

    return pl.kernel(body, out_type=out_type, mesh=mesh,
                     scratch_types=scratch,
                     compiler_params=pltpu.CompilerParams(
                         use_tc_tiling_on_sc=False,
                         needs_layout_passes=False))


def _proj1_body(x_ref, wl_ref, wr_ref, b_ref, xl_ref, xr_ref):
    xb = x_ref[...]
    xl_ref[...] = jnp.dot(xb, wl_ref[...], preferred_element_type=jnp.float32)
    xr_ref[...] = (jnp.dot(xb, wr_ref[...], preferred_element_type=jnp.float32)
                   + b_ref[...])


def _mid_body(aggp_ref, degp_ref, xr_ref, wl_ref, wr_ref, b_ref,
              hl_ref, hr_ref):
    deg = jnp.clip(degp_ref[0] + degp_ref[1], 1.0, None)
    agg = (aggp_ref[0] + aggp_ref[1]) / deg
    h = jnp.maximum(agg + xr_ref[...], 0.0)
    hl_ref[...] = jnp.dot(h, wl_ref[...], preferred_element_type=jnp.float32)
    hr_ref[...] = (jnp.dot(h, wr_ref[...], preferred_element_type=jnp.float32)
                   + b_ref[...])


def _out_body(aggp_ref, degp_ref, hr_ref, o_ref):
    deg = jnp.clip(degp_ref[0] + degp_ref[1], 1.0, None)
    z = (aggp_ref[0] + aggp_ref[1]) / deg + hr_ref[...]
    m = jnp.max(z, axis=1, keepdims=True)
    e = jnp.exp(z - m)
    o_ref[...] = (z - m) - jnp.log(jnp.sum(e, axis=1, keepdims=True))


def kernel(x, edge_index, edge_weight, W1_l, W1_r, b1, W3_l, W3_r, b3):
    n, fin = x.shape
    emb = W1_l.shape[0]
    n_edges = edge_weight.shape[0]
    blk_edges = NC * NS * B * K
    e_pad = -(-n_edges // blk_edges) * blk_edges
    rpt = -(-(n // NS) // 8) * 8
    n_pad = rpt * NS
    src = jnp.concatenate(
        [edge_index[0], jnp.zeros((e_pad - n_edges,), jnp.int32)])
    dst2 = jnp.concatenate(
        [edge_index[1],
         jnp.full((e_pad - n_edges,), n_pad - 1, jnp.int32)]).reshape(
             e_pad // K, K)
    ew = jnp.concatenate(
        [edge_weight, jnp.zeros((e_pad - n_edges,), jnp.float32)])
    zeros2 = jnp.zeros((n_pad, emb), jnp.float32)
    zeros1 = jnp.zeros((n_pad,), jnp.float32)
    grid = (n // BLK,)

    full = lambda shape: pl.BlockSpec(shape, lambda i: (0,) * len(shape))
    rows2 = pl.BlockSpec((BLK, fin), lambda i: (i, 0))
    rows_e = pl.BlockSpec((BLK, emb), lambda i: (i, 0))
    part3 = pl.BlockSpec((NC, BLK, emb), lambda i: (0, i, 0))
    part_deg = pl.BlockSpec((NC, BLK, 1), lambda i: (0, i, 0))
    f32 = lambda shape: jax.ShapeDtypeStruct(shape, jnp.float32)

    xl, xr = pl.pallas_call(
        _proj1_body, grid=grid,
        in_specs=[rows2, full((fin, emb)), full((fin, emb)), full((1, emb))],
        out_specs=[rows_e, rows_e],
        out_shape=[f32((n, emb)), f32((n, emb))],
    )(x, W1_l.T, W1_r.T, b1.reshape(1, emb))

    edge_pass_deg = _make_edge_pass(n, n_pad, emb, e_pad, with_deg=True)
    aggp1, degp = edge_pass_deg(xl, src, dst2, ew, zeros2, zeros1)
    degp3 = degp.reshape(NC, n_pad, 1)

    hl, hr = pl.pallas_call(
        _mid_body, grid=grid,
        in_specs=[part3, part_deg, rows_e,
                  full((emb, emb)), full((emb, emb)), full((1, emb))],
        out_specs=[rows_e, rows_e],
        out_shape=[f32((n, emb)), f32((n, emb))],
    )(aggp1, degp3, xr, W3_l.T, W3_r.T, b3.reshape(1, emb))

    edge_pass = _make_edge_pass(n, n_pad, emb, e_pad, with_deg=False)
    aggp2, = edge_pass(hl, src, dst2, ew, zeros2)

    out = pl.pallas_call(
        _out_body, grid=grid,
        in_specs=[part3, part_deg, rows_e],
        out_specs=rows_e,
        out_shape=f32((n, emb)),
    )(aggp2, degp3, hr)
    return out


# speedup vs baseline: 23.9315x; 1.1117x over previous
import jax
import jax.numpy as jnp
from jax import lax
from jax.experimental import pallas as pl
from jax.experimental.pallas import tpu as pltpu
from jax.experimental.pallas import tpu_sc as plsc

NC = 2
NS = 16
K = 128
B = 16
BLK = 4000


def _make_edge_pass(n_nodes, n_pad, emb, n_edges, with_deg):
    e_per_tile = n_edges // (NC * NS)
    assert e_per_tile * NC * NS == n_edges and e_per_tile % (B * K) == 0
    assert emb == 8 and 16 % emb == 0
    n_blocks = e_per_tile // (B * K)
    rows_per_tile = n_pad // NS
    assert rows_per_tile * NS == n_pad and rows_per_tile % 8 == 0

    mesh = plsc.VectorSubcoreMesh(
        core_axis_name="c", subcore_axis_name="s",
        num_cores=NC, num_subcores=NS)

    out_type = [jax.ShapeDtypeStruct((NC, n_pad, emb), jnp.float32)]
    scratch = [
        pltpu.VMEM((B * K,), jnp.int32),
        pltpu.VMEM((B, K), jnp.int32),
        pltpu.VMEM((B * K,), jnp.float32),
        pltpu.VMEM((K, emb), jnp.float32),
        pltpu.VMEM((K, emb), jnp.float32),
        pltpu.VMEM((K, emb), jnp.float32),
        pltpu.VMEM((K, emb), jnp.float32),
        pltpu.VMEM_SHARED((n_pad, emb), jnp.float32),
        pltpu.SemaphoreType.DMA,
        pltpu.SemaphoreType.DMA,
        pltpu.SemaphoreType.DMA,
        pltpu.SemaphoreType.DMA,
        pltpu.SemaphoreType.DMA,
    ]
    if with_deg:
        out_type.append(jax.ShapeDtypeStruct((NC, n_pad), jnp.float32))
        scratch.append(pltpu.VMEM((K,), jnp.float32))
        scratch.append(pltpu.VMEM_SHARED((n_pad,), jnp.float32))
        scratch.append(pltpu.SemaphoreType.DMA)

    def body(table, src, dst2, w, zeros2, *rest):
        if with_deg:
            (zeros1, agg_out, deg_out, src_blk, dst_blk, w_blk,
             rows_a, rows_b, sc_a, sc_b, acc_sh, bsem, gsem_a, gsem_b,
             ssem_a, ssem_b, ones_v, deg_sh, dsem) = rest
        else:
            (agg_out, src_blk, dst_blk, w_blk,
             rows_a, rows_b, sc_a, sc_b, acc_sh, bsem, gsem_a, gsem_b,
             ssem_a, ssem_b) = rest

        c = lax.axis_index("c")
        s = lax.axis_index("s")
        base_e = (c * NS + s) * e_per_tile
        r0 = s * rows_per_tile

        pltpu.sync_copy(zeros2.at[pl.ds(r0, rows_per_tile)],
                        acc_sh.at[pl.ds(r0, rows_per_tile)])
        if with_deg:
            @pl.when(s == 0)
            def _():
                pltpu.sync_copy(zeros1, deg_sh)
            for i in range(K // 16):
                ones_v[pl.ds(i * 16, 16)] = jnp.full((16,), 1.0, jnp.float32)
        plsc.subcore_barrier()

        rows = (rows_a, rows_b)
        scrows = (sc_a, sc_b)
        gsem = (gsem_a, gsem_b)
        ssem = (ssem_a, ssem_b)
        lane16 = lax.iota(jnp.int32, 16)

        @pl.loop(0, n_blocks)
        def _block(blk):
            e0 = base_e + blk * (B * K)
            row0 = lax.div(e0, K)
            ld_s = pltpu.async_copy(src.at[pl.ds(e0, B * K)], src_blk, bsem)
            ld_d = pltpu.async_copy(dst2.at[pl.ds(row0, B)], dst_blk, bsem)
            ld_w = pltpu.async_copy(w.at[pl.ds(e0, B * K)], w_blk, bsem)
            ld_s.wait(); ld_d.wait(); ld_w.wait()

            g_desc = [None, None]
            s_desc = [None, None]
            d_desc = [None]
            g_desc[0] = pltpu.async_copy(
                table.at[src_blk.at[pl.ds(0, K)]], rows[0], gsem[0])
            for b in range(B):
                p = b & 1
                q = 1 - p
                if b + 1 < B:
                    g_desc[q] = pltpu.async_copy(
                        table.at[src_blk.at[pl.ds((b + 1) * K, K)]],
                        rows[q], gsem[q])
                g_desc[p].wait()
                if s_desc[p] is not None:
                    s_desc[p].wait()
                for j in range(K // 16):
                    ridx = lane16 + (j * 16)
                    wv = w_blk[pl.ds(b * K + j * 16, 16)]
                    for col in range(emb):
                        cidx = jnp.full((16,), col, jnp.int32)
                        v = plsc.load_gather(rows[p], [ridx, cidx])
                        plsc.store_scatter(scrows[p], [ridx, cidx], v * wv)
                s_desc[p] = pltpu.async_copy(
                    scrows[p], acc_sh.at[dst_blk.at[b]], ssem[p], add=True)
                if with_deg:
                    if d_desc[0] is not None:
                        d_desc[0].wait()
                    d_desc[0] = pltpu.async_copy(
                        ones_v, deg_sh.at[dst_blk.at[b]], dsem, add=True)
            s_desc[0].wait()
            s_desc[1].wait()
            if with_deg:
                d_desc[0].wait()

        plsc.subcore_barrier()
        pltpu.sync_copy(acc_sh.at[pl.ds(r0, rows_per_tile)],
                        agg_out.at[c, pl.ds(r0, rows_per_tile)])
        if with_deg:
            @pl.when(s == 0)
            def _():
                pltpu.sync_copy(deg_sh, deg_out.at[c])

    return pl.kernel(body, out_type=out_type, mesh=mesh,
                     scratch_types=scratch,
                     compiler_params=pltpu.CompilerParams(
                         use_tc_tiling_on_sc=False,
                         needs_layout_passes=False))


def _proj1_body(x_ref, wl_ref, wr_ref, b_ref, xl_ref, xr_ref):
    xb = x_ref[...]
    xl_ref[...] = jnp.dot(xb, wl_ref[...], preferred_element_type=jnp.float32)
    xr_ref[...] = (jnp.dot(xb, wr_ref[...], preferred_element_type=jnp.float32)
                   + b_ref[...])


def _mid_body(aggp_ref, degp_ref, xr_ref, wl_ref, wr_ref, b_ref,
              hl_ref, hr_ref):
    deg = jnp.clip(degp_ref[0] + degp_ref[1], 1.0, None)
    agg = (aggp_ref[0] + aggp_ref[1]) / deg
    h = jnp.maximum(agg + xr_ref[...], 0.0)
    hl_ref[...] = jnp.dot(h, wl_ref[...], preferred_element_type=jnp.float32)
    hr_ref[...] = (jnp.dot(h, wr_ref[...], preferred_element_type=jnp.float32)
                   + b_ref[...])


def _out_body(aggp_ref, degp_ref, hr_ref, o_ref):
    deg = jnp.clip(degp_ref[0] + degp_ref[1], 1.0, None)
    z = (aggp_ref[0] + aggp_ref[1]) / deg + hr_ref[...]
    m = jnp.max(z, axis=1, keepdims=True)
    e = jnp.exp(z - m)
    o_ref[...] = (z - m) - jnp.log(jnp.sum(e, axis=1, keepdims=True))


def kernel(x, edge_index, edge_weight, W1_l, W1_r, b1, W3_l, W3_r, b3):
    n, fin = x.shape
    emb = W1_l.shape[0]
    n_edges = edge_weight.shape[0]
    blk_edges = NC * NS * B * K
    e_pad = -(-n_edges // blk_edges) * blk_edges
    rpt = -(-(n // NS) // 8) * 8
    n_pad = rpt * NS
    src = jnp.concatenate(
        [edge_index[0], jnp.zeros((e_pad - n_edges,), jnp.int32)])
    dst2 = jnp.concatenate(
        [edge_index[1],
         jnp.full((e_pad - n_edges,), n_pad - 1, jnp.int32)]).reshape(
             e_pad // K, K)
    ew = jnp.concatenate(
        [edge_weight, jnp.zeros((e_pad - n_edges,), jnp.float32)])
    zeros2 = jnp.zeros((n_pad, emb), jnp.float32)
    zeros1 = jnp.zeros((n_pad,), jnp.float32)
    grid = (n // BLK,)

    full = lambda shape: pl.BlockSpec(shape, lambda i: (0,) * len(shape))
    rows2 = pl.BlockSpec((BLK, fin), lambda i: (i, 0))
    rows_e = pl.BlockSpec((BLK, emb), lambda i: (i, 0))
    part3 = pl.BlockSpec((NC, BLK, emb), lambda i: (0, i, 0))
    part_deg = pl.BlockSpec((NC, BLK, 1), lambda i: (0, i, 0))
    f32 = lambda shape: jax.ShapeDtypeStruct(shape, jnp.float32)

    xl, xr = pl.pallas_call(
        _proj1_body, grid=grid,
        in_specs=[rows2, full((fin, emb)), full((fin, emb)), full((1, emb))],
        out_specs=[rows_e, rows_e],
        out_shape=[f32((n, emb)), f32((n, emb))],
    )(x, W1_l.T, W1_r.T, b1.reshape(1, emb))

    edge_pass_deg = _make_edge_pass(n, n_pad, emb, e_pad, with_deg=True)
    aggp1, degp = edge_pass_deg(xl, src, dst2, ew, zeros2, zeros1)
    degp3 = degp.reshape(NC, n_pad, 1)

    hl, hr = pl.pallas_call(
        _mid_body, grid=grid,
        in_specs=[part3, part_deg, rows_e,
                  full((emb, emb)), full((emb, emb)), full((1, emb))],
        out_specs=[rows_e, rows_e],
        out_shape=[f32((n, emb)), f32((n, emb))],
    )(aggp1, degp3, xr, W3_l.T, W3_r.T, b3.reshape(1, emb))

    edge_pass = _make_edge_pass(n, n_pad, emb, e_pad, with_deg=False)
    aggp2, = edge_pass(hl, src, dst2, ew, zeros2)

    out = pl.pallas_call(
        _out_body, grid=grid,
        in_specs=[part3, part_deg, rows_e],
        out_specs=rows_e,
        out_shape=f32((n, emb)),
    )(aggp2, degp3, hr)
    return out


# speedup vs baseline: 26.1684x; 1.0935x over previous
import jax
import jax.numpy as jnp
from jax import lax
from jax.experimental import pallas as pl
from jax.experimental.pallas import tpu as pltpu
from jax.experimental.pallas import tpu_sc as plsc

NC = 2
NS = 16
K = 128
B = 16
BLK = 4000


def _make_edge_pass(n_nodes, n_pad, emb, n_edges, with_deg):
    e_per_tile = n_edges // (NC * NS)
    assert e_per_tile * NC * NS == n_edges and e_per_tile % (B * K) == 0
    assert emb == 8 and 16 % emb == 0
    n_blocks = e_per_tile // (B * K)
    rows_per_tile = n_pad // NS
    assert rows_per_tile * NS == n_pad and rows_per_tile % 8 == 0

    mesh = plsc.VectorSubcoreMesh(
        core_axis_name="c", subcore_axis_name="s",
        num_cores=NC, num_subcores=NS)

    out_type = [jax.ShapeDtypeStruct((NC, n_pad, emb), jnp.float32)]
    scratch = [
        pltpu.VMEM((B * K,), jnp.int32),
        pltpu.VMEM((B, K), jnp.int32),
        pltpu.VMEM((B * K,), jnp.float32),
        pltpu.VMEM((K, emb), jnp.float32),
        pltpu.VMEM((K, emb), jnp.float32),
        pltpu.VMEM((K, emb), jnp.float32),
        pltpu.VMEM((K, emb), jnp.float32),
        pltpu.VMEM((K, emb), jnp.float32),
        pltpu.VMEM((K, emb), jnp.float32),
        pltpu.VMEM_SHARED((n_pad, emb), jnp.float32),
        pltpu.SemaphoreType.DMA,
        pltpu.SemaphoreType.DMA,
        pltpu.SemaphoreType.DMA,
        pltpu.SemaphoreType.DMA,
        pltpu.SemaphoreType.DMA,
        pltpu.SemaphoreType.DMA,
        pltpu.SemaphoreType.DMA,
    ]
    if with_deg:
        out_type.append(jax.ShapeDtypeStruct((NC, n_pad), jnp.float32))
        scratch.append(pltpu.VMEM((K,), jnp.float32))
        scratch.append(pltpu.VMEM_SHARED((n_pad,), jnp.float32))
        scratch.append(pltpu.SemaphoreType.DMA)

    def body(table, src, dst2, w, zeros2, *rest):
        if with_deg:
            (zeros1, agg_out, deg_out, src_blk, dst_blk, w_blk,
             rows_0, rows_1, rows_2, rows_3, sc_a, sc_b, acc_sh, bsem,
             gsem_0, gsem_1, gsem_2, gsem_3,
             ssem_a, ssem_b, ones_v, deg_sh, dsem) = rest
        else:
            (agg_out, src_blk, dst_blk, w_blk,
             rows_0, rows_1, rows_2, rows_3, sc_a, sc_b, acc_sh, bsem,
             gsem_0, gsem_1, gsem_2, gsem_3,
             ssem_a, ssem_b) = rest

        c = lax.axis_index("c")
        s = lax.axis_index("s")
        base_e = (c * NS + s) * e_per_tile
        r0 = s * rows_per_tile

        pltpu.sync_copy(zeros2.at[pl.ds(r0, rows_per_tile)],
                        acc_sh.at[pl.ds(r0, rows_per_tile)])
        if with_deg:
            @pl.when(s == 0)
            def _():
                pltpu.sync_copy(zeros1, deg_sh)
            for i in range(K // 16):
                ones_v[pl.ds(i * 16, 16)] = jnp.full((16,), 1.0, jnp.float32)
        plsc.subcore_barrier()

        rows = (rows_0, rows_1, rows_2, rows_3)
        scrows = (sc_a, sc_b)
        gsem = (gsem_0, gsem_1, gsem_2, gsem_3)
        ssem = (ssem_a, ssem_b)
        lane16 = lax.iota(jnp.int32, 16)

        @pl.loop(0, n_blocks)
        def _block(blk):
            e0 = base_e + blk * (B * K)
            row0 = lax.div(e0, K)
            ld_s = pltpu.async_copy(src.at[pl.ds(e0, B * K)], src_blk, bsem)
            ld_d = pltpu.async_copy(dst2.at[pl.ds(row0, B)], dst_blk, bsem)
            ld_w = pltpu.async_copy(w.at[pl.ds(e0, B * K)], w_blk, bsem)
            ld_s.wait(); ld_d.wait(); ld_w.wait()

            g_desc = [None, None, None, None]
            s_desc = [None, None]
            d_desc = [None]
            for pre in range(3):
                g_desc[pre] = pltpu.async_copy(
                    table.at[src_blk.at[pl.ds(pre * K, K)]],
                    rows[pre], gsem[pre])
            for b in range(B):
                g = b & 3
                p = b & 1
                if b + 3 < B:
                    g_desc[(b + 3) & 3] = pltpu.async_copy(
                        table.at[src_blk.at[pl.ds((b + 3) * K, K)]],
                        rows[(b + 3) & 3], gsem[(b + 3) & 3])
                g_desc[g].wait()
                if s_desc[p] is not None:
                    s_desc[p].wait()
                for j in range(K // 16):
                    ridx = lane16 + (j * 16)
                    wv = w_blk[pl.ds(b * K + j * 16, 16)]
                    for col in range(emb):
                        cidx = jnp.full((16,), col, jnp.int32)
                        v = plsc.load_gather(rows[g], [ridx, cidx])
                        plsc.store_scatter(scrows[p], [ridx, cidx], v * wv)
                s_desc[p] = pltpu.async_copy(
                    scrows[p], acc_sh.at[dst_blk.at[b]], ssem[p], add=True)
                if with_deg:
                    if d_desc[0] is not None:
                        d_desc[0].wait()
                    d_desc[0] = pltpu.async_copy(
                        ones_v, deg_sh.at[dst_blk.at[b]], dsem, add=True)
            s_desc[0].wait()
            s_desc[1].wait()
            if with_deg:
                d_desc[0].wait()

        plsc.subcore_barrier()
        pltpu.sync_copy(acc_sh.at[pl.ds(r0, rows_per_tile)],
                        agg_out.at[c, pl.ds(r0, rows_per_tile)])
        if with_deg:
            @pl.when(s == 0)
            def _():
                pltpu.sync_copy(deg_sh, deg_out.at[c])

    return pl.kernel(body, out_type=out_type, mesh=mesh,
                     scratch_types=scratch,
                     compiler_params=pltpu.CompilerParams(
                         use_tc_tiling_on_sc=False,
                         needs_layout_passes=False))


def _proj1_body(x_ref, wl_ref, wr_ref, b_ref, xl_ref, xr_ref):
    xb = x_ref[...]
    xl_ref[...] = jnp.dot(xb, wl_ref[...], preferred_element_type=jnp.float32)
    xr_ref[...] = (jnp.dot(xb, wr_ref[...], preferred_element_type=jnp.float32)
                   + b_ref[...])


def _mid_body(aggp_ref, degp_ref, xr_ref, wl_ref, wr_ref, b_ref,
              hl_ref, hr_ref):
    deg = jnp.clip(degp_ref[0] + degp_ref[1], 1.0, None)
    agg = (aggp_ref[0] + aggp_ref[1]) / deg
    h = jnp.maximum(agg + xr_ref[...], 0.0)
    hl_ref[...] = jnp.dot(h, wl_ref[...], preferred_element_type=jnp.float32)
    hr_ref[...] = (jnp.dot(h, wr_ref[...], preferred_element_type=jnp.float32)
                   + b_ref[...])


def _out_body(aggp_ref, degp_ref, hr_ref, o_ref):
    deg = jnp.clip(degp_ref[0] + degp_ref[1], 1.0, None)
    z = (aggp_ref[0] + aggp_ref[1]) / deg + hr_ref[...]
    m = jnp.max(z, axis=1, keepdims=True)
    e = jnp.exp(z - m)
    o_ref[...] = (z - m) - jnp.log(jnp.sum(e, axis=1, keepdims=True))


def kernel(x, edge_index, edge_weight, W1_l, W1_r, b1, W3_l, W3_r, b3):
    n, fin = x.shape
    emb = W1_l.shape[0]
    n_edges = edge_weight.shape[0]
    blk_edges = NC * NS * B * K
    e_pad = -(-n_edges // blk_edges) * blk_edges
    rpt = -(-(n // NS) // 8) * 8
    n_pad = rpt * NS
    src = jnp.concatenate(
        [edge_index[0], jnp.zeros((e_pad - n_edges,), jnp.int32)])
    dst2 = jnp.concatenate(
        [edge_index[1],
         jnp.full((e_pad - n_edges,), n_pad - 1, jnp.int32)]).reshape(
             e_pad // K, K)
    ew = jnp.concatenate(
        [edge_weight, jnp.zeros((e_pad - n_edges,), jnp.float32)])
    zeros2 = jnp.zeros((n_pad, emb), jnp.float32)
    zeros1 = jnp.zeros((n_pad,), jnp.float32)
    grid = (n // BLK,)

    full = lambda shape: pl.BlockSpec(shape, lambda i: (0,) * len(shape))
    rows2 = pl.BlockSpec((BLK, fin), lambda i: (i, 0))
    rows_e = pl.BlockSpec((BLK, emb), lambda i: (i, 0))
    part3 = pl.BlockSpec((NC, BLK, emb), lambda i: (0, i, 0))
    part_deg = pl.BlockSpec((NC, BLK, 1), lambda i: (0, i, 0))
    f32 = lambda shape: jax.ShapeDtypeStruct(shape, jnp.float32)

    xl, xr = pl.pallas_call(
        _proj1_body, grid=grid,
        in_specs=[rows2, full((fin, emb)), full((fin, emb)), full((1, emb))],
        out_specs=[rows_e, rows_e],
        out_shape=[f32((n, emb)), f32((n, emb))],
    )(x, W1_l.T, W1_r.T, b1.reshape(1, emb))

    edge_pass_deg = _make_edge_pass(n, n_pad, emb, e_pad, with_deg=True)
    aggp1, degp = edge_pass_deg(xl, src, dst2, ew, zeros2, zeros1)
    degp3 = degp.reshape(NC, n_pad, 1)

    hl, hr = pl.pallas_call(
        _mid_body, grid=grid,
        in_specs=[part3, part_deg, rows_e,
                  full((emb, emb)), full((emb, emb)), full((1, emb))],
        out_specs=[rows_e, rows_e],
        out_shape=[f32((n, emb)), f32((n, emb))],
    )(aggp1, degp3, xr, W3_l.T, W3_r.T, b3.reshape(1, emb))

    edge_pass = _make_edge_pass(n, n_pad, emb, e_pad, with_deg=False)
    aggp2, = edge_pass(hl, src, dst2, ew, zeros2)

    out = pl.pallas_call(
        _out_body, grid=grid,
        in_specs=[part3, part_deg, rows_e],
        out_specs=rows_e,
        out_shape=f32((n, emb)),
    )(aggp2, degp3, hr)
    return out
